# Initial kernel scaffold; baseline (speedup 1.0000x reference)
import jax
import jax.numpy as jnp
from jax.experimental import pallas as pl


def kernel(X, edge_index, W1, b1, W2, b2):
    raise NotImplementedError("write your pallas kernel here")


# speedup vs baseline: 10.5387x; 10.5387x over previous
import functools

import jax
import jax.numpy as jnp
from jax import lax
from jax.experimental import pallas as pl
from jax.experimental.pallas import tpu as pltpu
from jax.experimental.pallas import tpu_sc as plsc

N = 10000
NP = 10240
E = 320000
NC = 2
NS = 16
NW = NC * NS
EW = E // NW
K = 80
NCHUNK = EW // K
RPT = NP // NS
ZR = 128

_mesh = plsc.VectorSubcoreMesh(core_axis_name="c", subcore_axis_name="s",
                               num_cores=NC, num_subcores=NS)


def _degree_body(dst_hbm, out_hbm, didx, hist):
    cid = lax.axis_index("c")
    sid = lax.axis_index("s")
    wid = cid * NS + sid
    base = wid * EW

    @pl.loop(0, NP // 16)
    def _zero(i):
        hist[pl.ds(i * 16, 16)] = jnp.zeros((16,), jnp.float32)

    ones16 = jnp.ones((16,), jnp.float32)

    @pl.loop(0, NCHUNK)
    def _chunks(j):
        pltpu.sync_copy(dst_hbm.at[pl.ds(base + j * K, K)], didx)

        @pl.loop(0, K // 16)
        def _vec(i):
            dv = didx[pl.ds(i * 16, 16)]
            plsc.addupdate_scatter(hist, [dv], ones16)

    pltpu.sync_copy(hist, out_hbm.at[pl.ds(wid * NP, NP)])


def _degree_builder(interpret=False):
    return functools.partial(
        pl.kernel,
        out_type=jax.ShapeDtypeStruct((NW * NP,), jnp.float32),
        mesh=_mesh,
        interpret=interpret,
        compiler_params=pltpu.CompilerParams(needs_layout_passes=False),
        scratch_types=[
            pltpu.VMEM((K,), jnp.int32),
            pltpu.VMEM((NP,), jnp.float32),
        ],
    )(_degree_body)


_degree = _degree_builder()


def _scatter_body(g_hbm, src_hbm, dst_hbm, zeros_hbm, out_hbm,
                  sidx, didx, gbuf, acc, sem):
    cid = lax.axis_index("c")
    sid = lax.axis_index("s")
    base = (cid * NS + sid) * EW

    @pl.loop(0, RPT // ZR)
    def _zero(z):
        pltpu.sync_copy(zeros_hbm, acc.at[pl.ds(sid * RPT + z * ZR, ZR)])
    plsc.subcore_barrier()

    @pl.loop(0, NCHUNK)
    def _chunks(j):
        e0 = base + j * K
        pltpu.sync_copy(src_hbm.at[pl.ds(e0, K)], sidx)
        pltpu.sync_copy(dst_hbm.at[pl.ds(e0, K)], didx)
        pltpu.async_copy(g_hbm.at[sidx], gbuf, sem).wait()
        pltpu.sync_copy(gbuf, acc.at[didx], add=True)

    plsc.subcore_barrier()
    pltpu.sync_copy(acc.at[pl.ds(sid * RPT, RPT)],
                    out_hbm.at[cid, pl.ds(sid * RPT, RPT)])


def _scatter_builder(D, interpret=False):
    return functools.partial(
        pl.kernel,
        out_type=jax.ShapeDtypeStruct((NC, NP, D), jnp.float32),
        mesh=_mesh,
        interpret=interpret,
        scratch_types=[
            pltpu.VMEM((K,), jnp.int32),
            pltpu.VMEM((K,), jnp.int32),
            pltpu.VMEM((K, D), jnp.float32),
            pltpu.VMEM_SHARED((NP, D), jnp.float32),
            pltpu.SemaphoreType.DMA,
        ],
    )(_scatter_body)


_scatter128 = _scatter_builder(128)


def _tc1_body(x_ref, w1_ref, b1_ref, dinv_ref, g1_ref, s1_ref):
    dinv = dinv_ref[...]
    h = jnp.dot(x_ref[...], w1_ref[...],
                preferred_element_type=jnp.float32) + b1_ref[...]
    g1_ref[...] = h * dinv
    s1_ref[...] = h * (dinv * dinv)


def _tc2_body(acc_ref, s1_ref, dinv_ref, w2_ref, b2_ref,
              out1_ref, g2_ref, s2_ref):
    dinv = dinv_ref[...]
    out1 = jnp.maximum((acc_ref[0] + acc_ref[1]) * dinv + s1_ref[...], 0.0)
    out1_ref[...] = out1
    h2 = jnp.dot(out1, w2_ref[...],
                 preferred_element_type=jnp.float32) + b2_ref[...]
    g2_ref[...] = h2 * dinv
    s2_ref[...] = h2 * (dinv * dinv)


def _tc3_body(acc_ref, s2_ref, dinv_ref, out2_ref):
    dinv = dinv_ref[...]
    out2_ref[...] = (acc_ref[0] + acc_ref[1]) * dinv + s2_ref[...]


_f32 = jnp.float32
_tc1 = pl.pallas_call(
    _tc1_body,
    out_shape=(jax.ShapeDtypeStruct((NP, 128), _f32),
               jax.ShapeDtypeStruct((NP, 128), _f32)))
_tc2 = pl.pallas_call(
    _tc2_body,
    out_shape=(jax.ShapeDtypeStruct((NP, 128), _f32),
               jax.ShapeDtypeStruct((NP, 128), _f32),
               jax.ShapeDtypeStruct((NP, 128), _f32)))
_tc3 = pl.pallas_call(
    _tc3_body,
    out_shape=jax.ShapeDtypeStruct((NP, 128), _f32))


def kernel(X, edge_index, W1, b1, W2, b2):
    src = edge_index[0]
    dst = edge_index[1]
    zeros128 = jnp.zeros((ZR, 128), _f32)
    xp = jnp.pad(X, ((0, NP - N), (0, 0)))
    w2p = jnp.pad(W2, ((0, 0), (0, 128 - 64)))
    b2p = jnp.pad(b2, ((0, 128 - 64),)).reshape(1, -1)

    hist = _degree(dst)
    deg = hist.reshape(NW, NP).sum(axis=0) + 1.0
    dinv = lax.rsqrt(deg).reshape(NP, 1)

    g1, s1 = _tc1(xp, W1, b1.reshape(1, -1), dinv)
    acc1 = _scatter128(g1, src, dst, zeros128)
    out1, g2, s2 = _tc2(acc1, s1, dinv, w2p, b2p)
    acc2 = _scatter128(g2, src, dst, zeros128)
    out2 = _tc3(acc2, s2, dinv)
    return (out1[:N], out2[:N, :64])


# speedup vs baseline: 21.7030x; 2.0594x over previous
import functools

import jax
import jax.numpy as jnp
from jax import lax
from jax.experimental import pallas as pl
from jax.experimental.pallas import tpu as pltpu
from jax.experimental.pallas import tpu_sc as plsc

N = 10000
NP = 10240
E = 320000
NC = 2
NS = 16
NW = NC * NS
EW = E // NW
K = 80
NCHUNK = EW // K
RPT = NP // NS
ZR = 128

_mesh = plsc.VectorSubcoreMesh(core_axis_name="c", subcore_axis_name="s",
                               num_cores=NC, num_subcores=NS)


def _degree_body(dst_hbm, out_hbm, didx_all, hist):
    cid = lax.axis_index("c")
    sid = lax.axis_index("s")
    wid = cid * NS + sid
    base = wid * EW

    pltpu.sync_copy(dst_hbm.at[pl.ds(base, EW)], didx_all)

    @pl.loop(0, NP // 16)
    def _zero(i):
        hist[pl.ds(i * 16, 16)] = jnp.zeros((16,), jnp.float32)

    ones16 = jnp.ones((16,), jnp.float32)

    @pl.loop(0, EW // 16)
    def _vec(i):
        dv = didx_all[pl.ds(i * 16, 16)]
        plsc.addupdate_scatter(hist, [dv], ones16)

    pltpu.sync_copy(hist, out_hbm.at[pl.ds(wid * NP, NP)])


def _degree_builder(interpret=False):
    return functools.partial(
        pl.kernel,
        out_type=jax.ShapeDtypeStruct((NW * NP,), jnp.float32),
        mesh=_mesh,
        interpret=interpret,
        compiler_params=pltpu.CompilerParams(needs_layout_passes=False),
        scratch_types=[
            pltpu.VMEM((EW,), jnp.int32),
            pltpu.VMEM((NP,), jnp.float32),
        ],
    )(_degree_body)


_degree = _degree_builder()


NBUF = 4


def _scatter_body(g_hbm, src_hbm, dst_hbm, zeros_hbm, out_hbm,
                  sidx, didx, gbuf, acc, sem_s, sem_d, sem_g):
    cid = lax.axis_index("c")
    sid = lax.axis_index("s")
    base = (cid * NS + sid) * EW

    @pl.loop(0, RPT // ZR)
    def _zero(z):
        pltpu.sync_copy(zeros_hbm, acc.at[pl.ds(sid * RPT + z * ZR, ZR)])
    plsc.subcore_barrier()

    def start_idx(jb, b):
        e0 = base + jb * K
        pltpu.async_copy(src_hbm.at[pl.ds(e0, K)], sidx.at[b], sem_s.at[b])
        pltpu.async_copy(dst_hbm.at[pl.ds(e0, K)], didx.at[b], sem_d.at[b])

    def wait_idx(b):
        pltpu.make_async_copy(src_hbm.at[pl.ds(base, K)],
                              sidx.at[b], sem_s.at[b]).wait()
        pltpu.make_async_copy(dst_hbm.at[pl.ds(base, K)],
                              didx.at[b], sem_d.at[b]).wait()

    def start_gather(b):
        pltpu.async_copy(g_hbm.at[sidx.at[b]], gbuf.at[b], sem_g.at[b])

    def wait_gather_scatter(b):
        pltpu.make_async_copy(g_hbm.at[sidx.at[b]],
                              gbuf.at[b], sem_g.at[b]).wait()
        pltpu.sync_copy(gbuf.at[b], acc.at[didx.at[b]], add=True)

    for b in range(NBUF):
        start_idx(b, b)
    for b in range(NBUF):
        wait_idx(b)
        start_gather(b)

    @pl.loop(0, NCHUNK - 1, step=NBUF)
    def _chunks(j):
        for b in range(NBUF):
            jb = j + b
            wait_gather_scatter(b)

            @pl.when(jb + NBUF < NCHUNK - 1)
            def _next():
                start_idx(jb + NBUF, b)
                wait_idx(b)
                start_gather(b)

    start_idx(NCHUNK - 1, 0)
    wait_idx(0)
    start_gather(0)
    wait_gather_scatter(0)

    plsc.subcore_barrier()
    pltpu.sync_copy(acc.at[pl.ds(sid * RPT, RPT)],
                    out_hbm.at[cid, pl.ds(sid * RPT, RPT)])


def _scatter_builder(D, interpret=False):
    return functools.partial(
        pl.kernel,
        out_type=jax.ShapeDtypeStruct((NC, NP, D), jnp.float32),
        mesh=_mesh,
        interpret=interpret,
        scratch_types=[
            pltpu.VMEM((NBUF, K), jnp.int32),
            pltpu.VMEM((NBUF, K), jnp.int32),
            pltpu.VMEM((NBUF, K, D), jnp.float32),
            pltpu.VMEM_SHARED((NP, D), jnp.float32),
            pltpu.SemaphoreType.DMA((NBUF,)),
            pltpu.SemaphoreType.DMA((NBUF,)),
            pltpu.SemaphoreType.DMA((NBUF,)),
        ],
    )(_scatter_body)


_scatter128 = _scatter_builder(128)


def _tc1_body(x_ref, w1_ref, b1_ref, dinv_ref, g1_ref, s1_ref):
    dinv = dinv_ref[...]
    h = jnp.dot(x_ref[...], w1_ref[...],
                preferred_element_type=jnp.float32) + b1_ref[...]
    g1_ref[...] = h * dinv
    s1_ref[...] = h * (dinv * dinv)


def _tc2_body(acc_ref, s1_ref, dinv_ref, w2_ref, b2_ref,
              out1_ref, g2_ref, s2_ref):
    dinv = dinv_ref[...]
    out1 = jnp.maximum((acc_ref[0] + acc_ref[1]) * dinv + s1_ref[...], 0.0)
    out1_ref[...] = out1
    h2 = jnp.dot(out1, w2_ref[...],
                 preferred_element_type=jnp.float32) + b2_ref[...]
    g2_ref[...] = h2 * dinv
    s2_ref[...] = h2 * (dinv * dinv)


def _tc3_body(acc_ref, s2_ref, dinv_ref, out2_ref):
    dinv = dinv_ref[...]
    out2_ref[...] = (acc_ref[0] + acc_ref[1]) * dinv + s2_ref[...]


_f32 = jnp.float32
_tc1 = pl.pallas_call(
    _tc1_body,
    out_shape=(jax.ShapeDtypeStruct((NP, 128), _f32),
               jax.ShapeDtypeStruct((NP, 128), _f32)))
_tc2 = pl.pallas_call(
    _tc2_body,
    out_shape=(jax.ShapeDtypeStruct((NP, 128), _f32),
               jax.ShapeDtypeStruct((NP, 128), _f32),
               jax.ShapeDtypeStruct((NP, 128), _f32)))
_tc3 = pl.pallas_call(
    _tc3_body,
    out_shape=jax.ShapeDtypeStruct((NP, 128), _f32))


def kernel(X, edge_index, W1, b1, W2, b2):
    src = edge_index[0]
    dst = edge_index[1]
    zeros128 = jnp.zeros((ZR, 128), _f32)
    xp = jnp.pad(X, ((0, NP - N), (0, 0)))
    w2p = jnp.pad(W2, ((0, 0), (0, 128 - 64)))
    b2p = jnp.pad(b2, ((0, 128 - 64),)).reshape(1, -1)

    hist = _degree(dst)
    deg = hist.reshape(NW, NP).sum(axis=0) + 1.0
    dinv = lax.rsqrt(deg).reshape(NP, 1)

    g1, s1 = _tc1(xp, W1, b1.reshape(1, -1), dinv)
    acc1 = _scatter128(g1, src, dst, zeros128)
    out1, g2, s2 = _tc2(acc1, s1, dinv, w2p, b2p)
    acc2 = _scatter128(g2, src, dst, zeros128)
    out2 = _tc3(acc2, s2, dinv)
    return (out1[:N], out2[:N, :64])


# speedup vs baseline: 28.0173x; 1.2909x over previous
import functools

import jax
import jax.numpy as jnp
from jax import lax
from jax.experimental import pallas as pl
from jax.experimental.pallas import tpu as pltpu
from jax.experimental.pallas import tpu_sc as plsc

N = 10000
NP = 10240
E = 320000
NC = 2
NS = 16
NW = NC * NS
EW = E // NW
K = 80
NCHUNK = EW // K
RPT = NP // NS
ZR = 128

_mesh = plsc.VectorSubcoreMesh(core_axis_name="c", subcore_axis_name="s",
                               num_cores=NC, num_subcores=NS)


def _degree_body(dst_hbm, out_hbm, didx_all, hist):
    cid = lax.axis_index("c")
    sid = lax.axis_index("s")
    wid = cid * NS + sid
    base = wid * EW

    pltpu.sync_copy(dst_hbm.at[pl.ds(base, EW)], didx_all)

    @pl.loop(0, NP // 16)
    def _zero(i):
        hist[pl.ds(i * 16, 16)] = jnp.zeros((16,), jnp.float32)

    ones16 = jnp.ones((16,), jnp.float32)

    @pl.loop(0, EW // 16)
    def _vec(i):
        dv = didx_all[pl.ds(i * 16, 16)]
        plsc.addupdate_scatter(hist, [dv], ones16)

    pltpu.sync_copy(hist, out_hbm.at[pl.ds(wid * NP, NP)])


def _degree_builder(interpret=False):
    return functools.partial(
        pl.kernel,
        out_type=jax.ShapeDtypeStruct((NW * NP,), jnp.float32),
        mesh=_mesh,
        interpret=interpret,
        compiler_params=pltpu.CompilerParams(needs_layout_passes=False),
        scratch_types=[
            pltpu.VMEM((EW,), jnp.int32),
            pltpu.VMEM((NP,), jnp.float32),
        ],
    )(_degree_body)


_degree = _degree_builder()


NBUF = 4
IBUF = 8


def _scatter_body(g_hbm, src_hbm, dst_hbm, zeros_hbm, out_hbm,
                  sidx, didx, gbuf, acc, sem_s, sem_d, sem_g):
    cid = lax.axis_index("c")
    sid = lax.axis_index("s")
    base = (cid * NS + sid) * EW

    @pl.loop(0, RPT // ZR)
    def _zero(z):
        pltpu.sync_copy(zeros_hbm, acc.at[pl.ds(sid * RPT + z * ZR, ZR)])
    plsc.subcore_barrier()

    def start_idx(jb, i):
        e0 = base + jb * K
        pltpu.async_copy(src_hbm.at[pl.ds(e0, K)], sidx.at[i], sem_s.at[i])
        pltpu.async_copy(dst_hbm.at[pl.ds(e0, K)], didx.at[i], sem_d.at[i])

    def wait_idx(i):
        pltpu.make_async_copy(src_hbm.at[pl.ds(base, K)],
                              sidx.at[i], sem_s.at[i]).wait()
        pltpu.make_async_copy(dst_hbm.at[pl.ds(base, K)],
                              didx.at[i], sem_d.at[i]).wait()

    def start_gather(i, b):
        pltpu.async_copy(g_hbm.at[sidx.at[i]], gbuf.at[b], sem_g.at[b])

    def wait_gather_scatter(i, b):
        pltpu.make_async_copy(g_hbm.at[sidx.at[i]],
                              gbuf.at[b], sem_g.at[b]).wait()
        pltpu.sync_copy(gbuf.at[b], acc.at[didx.at[i]], add=True)

    for i in range(IBUF):
        start_idx(i, i)
    for b in range(NBUF):
        wait_idx(b)
        start_gather(b, b)

    @pl.loop(0, NCHUNK - NBUF - 1, step=IBUF)
    def _chunks(j):
        for b in range(IBUF):
            jb = j + b
            gb = b % NBUF
            wait_gather_scatter(b, gb)

            @pl.when(jb + IBUF < NCHUNK)
            def _pref():
                start_idx(jb + IBUF, b)

            i4 = (b + NBUF) % IBUF
            wait_idx(i4)
            start_gather(i4, gb)

    for c in range(NCHUNK - NBUF - 1, NCHUNK):
        i = c % IBUF
        gb = c % NBUF
        wait_gather_scatter(i, gb)
        nxt = c + NBUF
        if nxt < NCHUNK:
            wait_idx(nxt % IBUF)
            start_gather(nxt % IBUF, nxt % NBUF)

    plsc.subcore_barrier()
    pltpu.sync_copy(acc.at[pl.ds(sid * RPT, RPT)],
                    out_hbm.at[cid, pl.ds(sid * RPT, RPT)])


def _scatter_builder(D, interpret=False):
    return functools.partial(
        pl.kernel,
        out_type=jax.ShapeDtypeStruct((NC, NP, D), jnp.float32),
        mesh=_mesh,
        interpret=interpret,
        scratch_types=[
            pltpu.VMEM((IBUF, K), jnp.int32),
            pltpu.VMEM((IBUF, K), jnp.int32),
            pltpu.VMEM((NBUF, K, D), jnp.float32),
            pltpu.VMEM_SHARED((NP, D), jnp.float32),
            pltpu.SemaphoreType.DMA((IBUF,)),
            pltpu.SemaphoreType.DMA((IBUF,)),
            pltpu.SemaphoreType.DMA((NBUF,)),
        ],
    )(_scatter_body)


_scatter128 = _scatter_builder(128)


def _tc1_body(x_ref, w1_ref, b1_ref, dinv_ref, g1_ref, s1_ref):
    dinv = dinv_ref[...]
    h = jnp.dot(x_ref[...], w1_ref[...],
                preferred_element_type=jnp.float32) + b1_ref[...]
    g1_ref[...] = h * dinv
    s1_ref[...] = h * (dinv * dinv)


def _tc2_body(acc_ref, s1_ref, dinv_ref, w2_ref, b2_ref,
              out1_ref, g2_ref, s2_ref):
    dinv = dinv_ref[...]
    out1 = jnp.maximum((acc_ref[0] + acc_ref[1]) * dinv + s1_ref[...], 0.0)
    out1_ref[...] = out1
    h2 = jnp.dot(out1, w2_ref[...],
                 preferred_element_type=jnp.float32) + b2_ref[...]
    g2_ref[...] = h2 * dinv
    s2_ref[...] = h2 * (dinv * dinv)


def _tc3_body(acc_ref, s2_ref, dinv_ref, out2_ref):
    dinv = dinv_ref[...]
    out2_ref[...] = (acc_ref[0] + acc_ref[1]) * dinv + s2_ref[...]


_f32 = jnp.float32
_tc1 = pl.pallas_call(
    _tc1_body,
    out_shape=(jax.ShapeDtypeStruct((NP, 128), _f32),
               jax.ShapeDtypeStruct((NP, 128), _f32)))
_tc2 = pl.pallas_call(
    _tc2_body,
    out_shape=(jax.ShapeDtypeStruct((NP, 128), _f32),
               jax.ShapeDtypeStruct((NP, 128), _f32),
               jax.ShapeDtypeStruct((NP, 128), _f32)))
_tc3 = pl.pallas_call(
    _tc3_body,
    out_shape=jax.ShapeDtypeStruct((NP, 128), _f32))


def kernel(X, edge_index, W1, b1, W2, b2):
    src = edge_index[0]
    dst = edge_index[1]
    zeros128 = jnp.zeros((ZR, 128), _f32)
    xp = jnp.pad(X, ((0, NP - N), (0, 0)))
    w2p = jnp.pad(W2, ((0, 0), (0, 128 - 64)))
    b2p = jnp.pad(b2, ((0, 128 - 64),)).reshape(1, -1)

    hist = _degree(dst)
    deg = hist.reshape(NW, NP).sum(axis=0) + 1.0
    dinv = lax.rsqrt(deg).reshape(NP, 1)

    g1, s1 = _tc1(xp, W1, b1.reshape(1, -1), dinv)
    acc1 = _scatter128(g1, src, dst, zeros128)
    out1, g2, s2 = _tc2(acc1, s1, dinv, w2p, b2p)
    acc2 = _scatter128(g2, src, dst, zeros128)
    out2 = _tc3(acc2, s2, dinv)
    return (out1[:N], out2[:N, :64])


# speedup vs baseline: 28.0718x; 1.0019x over previous
import functools

import jax
import jax.numpy as jnp
from jax import lax
from jax.experimental import pallas as pl
from jax.experimental.pallas import tpu as pltpu
from jax.experimental.pallas import tpu_sc as plsc

N = 10000
NP = 10240
E = 320000
NC = 2
NS = 16
NW = NC * NS
EW = E // NW
K = 80
NCHUNK = EW // K
RPT = NP // NS
ZR = 128

_mesh = plsc.VectorSubcoreMesh(core_axis_name="c", subcore_axis_name="s",
                               num_cores=NC, num_subcores=NS)


def _degree_body(dst_hbm, out_hbm, didx_all, hist):
    cid = lax.axis_index("c")
    sid = lax.axis_index("s")
    wid = cid * NS + sid
    base = wid * EW

    pltpu.sync_copy(dst_hbm.at[pl.ds(base, EW)], didx_all)

    @pl.loop(0, NP // 16)
    def _zero(i):
        hist[pl.ds(i * 16, 16)] = jnp.zeros((16,), jnp.float32)

    ones16 = jnp.ones((16,), jnp.float32)

    @pl.loop(0, EW // 16)
    def _vec(i):
        dv = didx_all[pl.ds(i * 16, 16)]
        plsc.addupdate_scatter(hist, [dv], ones16)

    pltpu.sync_copy(hist, out_hbm.at[pl.ds(wid * NP, NP)])


def _degree_builder(interpret=False):
    return functools.partial(
        pl.kernel,
        out_type=jax.ShapeDtypeStruct((NW * NP,), jnp.float32),
        mesh=_mesh,
        interpret=interpret,
        compiler_params=pltpu.CompilerParams(needs_layout_passes=False),
        scratch_types=[
            pltpu.VMEM((EW,), jnp.int32),
            pltpu.VMEM((NP,), jnp.float32),
        ],
    )(_degree_body)


_degree = _degree_builder()


NBUF = 4
IBUF = 8
IPRE = 8
GDIST = 3
UNROLL = 8


def _scatter_body(g_hbm, src_hbm, dst_hbm, zeros_hbm, out_hbm,
                  sidx, didx, gbuf, acc, sem_s, sem_d, sem_g):
    cid = lax.axis_index("c")
    sid = lax.axis_index("s")
    base = (cid * NS + sid) * EW

    @pl.loop(0, RPT // ZR)
    def _zero(z):
        pltpu.sync_copy(zeros_hbm, acc.at[pl.ds(sid * RPT + z * ZR, ZR)])
    plsc.subcore_barrier()

    def start_idx(jb, i):
        e0 = base + jb * K
        pltpu.async_copy(src_hbm.at[pl.ds(e0, K)], sidx.at[i], sem_s.at[i])
        pltpu.async_copy(dst_hbm.at[pl.ds(e0, K)], didx.at[i], sem_d.at[i])

    def wait_idx(i):
        pltpu.make_async_copy(src_hbm.at[pl.ds(base, K)],
                              sidx.at[i], sem_s.at[i]).wait()
        pltpu.make_async_copy(dst_hbm.at[pl.ds(base, K)],
                              didx.at[i], sem_d.at[i]).wait()

    def start_gather(i, b):
        pltpu.async_copy(g_hbm.at[sidx.at[i]], gbuf.at[b], sem_g.at[b])

    def wait_gather(i, b):
        pltpu.make_async_copy(g_hbm.at[sidx.at[i]],
                              gbuf.at[b], sem_g.at[b]).wait()

    def scatter(i, b):
        pltpu.sync_copy(gbuf.at[b], acc.at[didx.at[i]], add=True)

    for i in range(IPRE):
        start_idx(i, i)
    for b in range(GDIST):
        wait_idx(b)
        start_gather(b, b)

    def process(jb, i, gb, prefetch_pred=None, launch_next=True):
        wait_gather(i, gb)
        if launch_next:
            nxt_i = (i + GDIST) % IBUF
            wait_idx(nxt_i)
            start_gather(nxt_i, (i + GDIST) % NBUF)
        scatter(i, gb)
        if prefetch_pred is True:
            start_idx(jb + IPRE, i)
        elif prefetch_pred is not None:
            @pl.when(prefetch_pred)
            def _pref():
                start_idx(jb + IPRE, i)

    NMAIN = NCHUNK // UNROLL * UNROLL

    @pl.loop(0, NMAIN, step=UNROLL)
    def _chunks(j):
        for b in range(UNROLL):
            process(j + b, b, b % NBUF, prefetch_pred=j + b + IPRE < NCHUNK)

    for c in range(NMAIN, NCHUNK):
        process(c, c % IBUF, c % NBUF,
                prefetch_pred=None,
                launch_next=c + GDIST < NCHUNK)

    plsc.subcore_barrier()
    pltpu.sync_copy(acc.at[pl.ds(sid * RPT, RPT)],
                    out_hbm.at[cid, pl.ds(sid * RPT, RPT)])


def _scatter_builder(D, interpret=False):
    return functools.partial(
        pl.kernel,
        out_type=jax.ShapeDtypeStruct((NC, NP, D), jnp.float32),
        mesh=_mesh,
        interpret=interpret,
        scratch_types=[
            pltpu.VMEM((IBUF, K), jnp.int32),
            pltpu.VMEM((IBUF, K), jnp.int32),
            pltpu.VMEM((NBUF, K, D), jnp.float32),
            pltpu.VMEM_SHARED((NP, D), jnp.float32),
            pltpu.SemaphoreType.DMA((IBUF,)),
            pltpu.SemaphoreType.DMA((IBUF,)),
            pltpu.SemaphoreType.DMA((NBUF,)),
        ],
    )(_scatter_body)


_scatter128 = _scatter_builder(128)


def _tc1_body(x_ref, w1_ref, b1_ref, dinv_ref, g1_ref, s1_ref):
    dinv = dinv_ref[...]
    h = jnp.dot(x_ref[...], w1_ref[...],
                preferred_element_type=jnp.float32) + b1_ref[...]
    g1_ref[...] = h * dinv
    s1_ref[...] = h * (dinv * dinv)


def _tc2_body(acc_ref, s1_ref, dinv_ref, w2_ref, b2_ref,
              out1_ref, g2_ref, s2_ref):
    dinv = dinv_ref[...]
    out1 = jnp.maximum((acc_ref[0] + acc_ref[1]) * dinv + s1_ref[...], 0.0)
    out1_ref[...] = out1
    h2 = jnp.dot(out1, w2_ref[...],
                 preferred_element_type=jnp.float32) + b2_ref[...]
    g2_ref[...] = h2 * dinv
    s2_ref[...] = h2 * (dinv * dinv)


def _tc3_body(acc_ref, s2_ref, dinv_ref, out2_ref):
    dinv = dinv_ref[...]
    out2_ref[...] = (acc_ref[0] + acc_ref[1]) * dinv + s2_ref[...]


_f32 = jnp.float32
_tc1 = pl.pallas_call(
    _tc1_body,
    out_shape=(jax.ShapeDtypeStruct((NP, 128), _f32),
               jax.ShapeDtypeStruct((NP, 128), _f32)))
_tc2 = pl.pallas_call(
    _tc2_body,
    out_shape=(jax.ShapeDtypeStruct((NP, 128), _f32),
               jax.ShapeDtypeStruct((NP, 128), _f32),
               jax.ShapeDtypeStruct((NP, 128), _f32)))
_tc3 = pl.pallas_call(
    _tc3_body,
    out_shape=jax.ShapeDtypeStruct((NP, 128), _f32))


def kernel(X, edge_index, W1, b1, W2, b2):
    src = edge_index[0]
    dst = edge_index[1]
    zeros128 = jnp.zeros((ZR, 128), _f32)
    xp = jnp.pad(X, ((0, NP - N), (0, 0)))
    w2p = jnp.pad(W2, ((0, 0), (0, 128 - 64)))
    b2p = jnp.pad(b2, ((0, 128 - 64),)).reshape(1, -1)

    hist = _degree(dst)
    deg = hist.reshape(NW, NP).sum(axis=0) + 1.0
    dinv = lax.rsqrt(deg).reshape(NP, 1)

    g1, s1 = _tc1(xp, W1, b1.reshape(1, -1), dinv)
    acc1 = _scatter128(g1, src, dst, zeros128)
    out1, g2, s2 = _tc2(acc1, s1, dinv, w2p, b2p)
    acc2 = _scatter128(g2, src, dst, zeros128)
    out2 = _tc3(acc2, s2, dinv)
    return (out1[:N], out2[:N, :64])


# speedup vs baseline: 28.2911x; 1.0078x over previous
import functools

import jax
import jax.numpy as jnp
from jax import lax
from jax.experimental import pallas as pl
from jax.experimental.pallas import tpu as pltpu
from jax.experimental.pallas import tpu_sc as plsc

N = 10000
NP = 10240
E = 320000
NC = 2
NS = 16
NW = NC * NS
EW = E // NW
K = 80
NCHUNK = EW // K
RPT = NP // NS
ZR = 128

_mesh = plsc.VectorSubcoreMesh(core_axis_name="c", subcore_axis_name="s",
                               num_cores=NC, num_subcores=NS)


def _degree_body(dst_hbm, out_hbm, didx_all, hist):
    cid = lax.axis_index("c")
    sid = lax.axis_index("s")
    wid = cid * NS + sid
    base = wid * EW

    pltpu.sync_copy(dst_hbm.at[pl.ds(base, EW)], didx_all)

    @pl.loop(0, NP // 16)
    def _zero(i):
        hist[pl.ds(i * 16, 16)] = jnp.zeros((16,), jnp.float32)

    ones16 = jnp.ones((16,), jnp.float32)

    @pl.loop(0, EW // 16)
    def _vec(i):
        dv = didx_all[pl.ds(i * 16, 16)]
        plsc.addupdate_scatter(hist, [dv], ones16)

    pltpu.sync_copy(hist, out_hbm.at[pl.ds(wid * NP, NP)])


def _degree_builder(interpret=False):
    return functools.partial(
        pl.kernel,
        out_type=jax.ShapeDtypeStruct((NW * NP,), jnp.float32),
        mesh=_mesh,
        interpret=interpret,
        compiler_params=pltpu.CompilerParams(needs_layout_passes=False),
        scratch_types=[
            pltpu.VMEM((EW,), jnp.int32),
            pltpu.VMEM((NP,), jnp.float32),
        ],
    )(_degree_body)


_degree = _degree_builder()


NBUF = 4
IBUF = 8
IPRE = 8
GDIST = 3
UNROLL = 8


def _scatter_body(g_hbm, src_hbm, dst_hbm, zeros_hbm, out_hbm,
                  sidx, didx, gbuf, acc, sem_s, sem_d, sem_g):
    cid = lax.axis_index("c")
    sid = lax.axis_index("s")
    base = (cid * NS + sid) * EW

    @pl.loop(0, RPT // ZR)
    def _zero(z):
        pltpu.sync_copy(zeros_hbm, acc.at[pl.ds(sid * RPT + z * ZR, ZR)])
    plsc.subcore_barrier()

    def start_idx(jb, i):
        e0 = base + jb * K
        pltpu.async_copy(src_hbm.at[pl.ds(e0, K)], sidx.at[i], sem_s.at[i])
        pltpu.async_copy(dst_hbm.at[pl.ds(e0, K)], didx.at[i], sem_d.at[i])

    def wait_idx(i):
        pltpu.make_async_copy(src_hbm.at[pl.ds(base, K)],
                              sidx.at[i], sem_s.at[i]).wait()
        pltpu.make_async_copy(dst_hbm.at[pl.ds(base, K)],
                              didx.at[i], sem_d.at[i]).wait()

    def start_gather(i, b):
        pltpu.async_copy(g_hbm.at[sidx.at[i]], gbuf.at[b], sem_g.at[b])

    def wait_gather(i, b):
        pltpu.make_async_copy(g_hbm.at[sidx.at[i]],
                              gbuf.at[b], sem_g.at[b]).wait()

    def scatter(i, b):
        pltpu.sync_copy(gbuf.at[b], acc.at[didx.at[i]], add=True)

    for i in range(IPRE):
        start_idx(i, i)
    for b in range(GDIST):
        wait_idx(b)
        start_gather(b, b)

    def process(jb, i, gb, prefetch_pred=None, launch_next=True):
        wait_gather(i, gb)
        if launch_next:
            nxt_i = (i + GDIST) % IBUF
            wait_idx(nxt_i)
            start_gather(nxt_i, (i + GDIST) % NBUF)
        scatter(i, gb)
        if prefetch_pred is True:
            start_idx(jb + IPRE, i)
        elif prefetch_pred is not None:
            @pl.when(prefetch_pred)
            def _pref():
                start_idx(jb + IPRE, i)

    NMAIN = NCHUNK // UNROLL * UNROLL

    @pl.loop(0, NMAIN, step=UNROLL)
    def _chunks(j):
        for b in range(UNROLL):
            process(j + b, b, b % NBUF, prefetch_pred=j + b + IPRE < NCHUNK)

    for c in range(NMAIN, NCHUNK):
        process(c, c % IBUF, c % NBUF,
                prefetch_pred=None,
                launch_next=c + GDIST < NCHUNK)

    plsc.subcore_barrier()
    pltpu.sync_copy(acc.at[pl.ds(sid * RPT, RPT)],
                    out_hbm.at[cid, pl.ds(sid * RPT, RPT)])


def _scatter_builder(D, interpret=False):
    return functools.partial(
        pl.kernel,
        out_type=jax.ShapeDtypeStruct((NC, NP, D), jnp.float32),
        mesh=_mesh,
        interpret=interpret,
        scratch_types=[
            pltpu.VMEM((IBUF, K), jnp.int32),
            pltpu.VMEM((IBUF, K), jnp.int32),
            pltpu.VMEM((NBUF, K, D), jnp.float32),
            pltpu.VMEM_SHARED((NP, D), jnp.float32),
            pltpu.SemaphoreType.DMA((IBUF,)),
            pltpu.SemaphoreType.DMA((IBUF,)),
            pltpu.SemaphoreType.DMA((NBUF,)),
        ],
    )(_scatter_body)


_scatter128 = _scatter_builder(128)


def _tc1_body(x_ref, w1_ref, b1_ref, dinv_ref, g1_ref, s1_ref):
    dinv = dinv_ref[...]
    h = jnp.dot(x_ref[...], w1_ref[...],
                preferred_element_type=jnp.float32) + b1_ref[...]
    g1_ref[:N] = h * dinv
    s1_ref[...] = h * (dinv * dinv)


def _tc2_body(acc_ref, s1_ref, dinv_ref, w2_ref, b2_ref,
              out1_ref, g2_ref, s2_ref):
    dinv = dinv_ref[...]
    out1 = jnp.maximum(
        (acc_ref[0, :N, :] + acc_ref[1, :N, :]) * dinv + s1_ref[...], 0.0)
    out1_ref[...] = out1
    h2 = jnp.dot(out1, w2_ref[...],
                 preferred_element_type=jnp.float32) + b2_ref[...]
    g2_ref[:N, :64] = h2 * dinv
    s2_ref[...] = h2 * (dinv * dinv)


def _tc3_body(acc_ref, s2_ref, dinv_ref, out2_ref):
    dinv = dinv_ref[...]
    out2_ref[...] = ((acc_ref[0, :N, :64] + acc_ref[1, :N, :64]) * dinv
                     + s2_ref[...])


_f32 = jnp.float32
_tc1 = pl.pallas_call(
    _tc1_body,
    out_shape=(jax.ShapeDtypeStruct((NP, 128), _f32),
               jax.ShapeDtypeStruct((N, 128), _f32)))
_tc2 = pl.pallas_call(
    _tc2_body,
    out_shape=(jax.ShapeDtypeStruct((N, 128), _f32),
               jax.ShapeDtypeStruct((NP, 128), _f32),
               jax.ShapeDtypeStruct((N, 64), _f32)))
_tc3 = pl.pallas_call(
    _tc3_body,
    out_shape=jax.ShapeDtypeStruct((N, 64), _f32))


def kernel(X, edge_index, W1, b1, W2, b2):
    src = edge_index[0]
    dst = edge_index[1]
    zeros128 = jnp.zeros((ZR, 128), _f32)

    hist = _degree(dst)
    deg = hist.reshape(NW, NP)[:, :N].sum(axis=0) + 1.0
    dinv = lax.rsqrt(deg).reshape(N, 1)

    g1, s1 = _tc1(X, W1, b1.reshape(1, -1), dinv)
    acc1 = _scatter128(g1, src, dst, zeros128)
    out1, g2, s2 = _tc2(acc1, s1, dinv, W2, b2.reshape(1, -1))
    acc2 = _scatter128(g2, src, dst, zeros128)
    out2 = _tc3(acc2, s2, dinv)
    return (out1, out2)
